# Initial kernel scaffold; baseline (speedup 1.0000x reference)
import jax
import jax.numpy as jnp
from jax.experimental import pallas as pl


def kernel(x, slot_memory, slot_keys, W_in, ln_in_g, ln_in_b, W_write, Wg, bg, W_out, ln_s_g, ln_s_b):
    raise NotImplementedError("write your pallas kernel here")


# speedup vs baseline: 13.4139x; 13.4139x over previous
import functools

import jax
import jax.numpy as jnp
from jax import lax
from jax.experimental import pallas as pl
from jax.experimental.pallas import tpu as pltpu
from jax.experimental.pallas import tpu_sc as plsc

NUM_SLOTS = 64
SLOT_DIM = 128
NUM_HEADS = 4
INPUT_DIM = 2048
CHUNK = 16
TOP_K = 3
SEQ_TILE = 512


def _dense_body(x_ref, lng_ref, lnb_ref, win_ref, slots_ref, wout_ref,
                out_ref, cs_ref):
    hd = SLOT_DIM // NUM_HEADS
    scale = hd ** (-0.5)
    xb = x_ref[0]
    m = jnp.mean(xb, axis=-1, keepdims=True)
    xc = xb - m
    v = jnp.mean(xc * xc, axis=-1, keepdims=True)
    xn = xc * lax.rsqrt(v + 1e-5) * lng_ref[...] + lnb_ref[...]
    xp = jnp.dot(xn, win_ref[...], preferred_element_type=jnp.float32)

    parts = []
    for h in range(NUM_HEADS):
        kh = slots_ref[:, h * hd:(h + 1) * hd]
        qh = xp[:, h * hd:(h + 1) * hd]
        sh = lax.dot_general(qh, kh, (((1,), (1,)), ((), ())),
                             preferred_element_type=jnp.float32) * scale
        mx = jnp.max(sh, axis=-1, keepdims=True)
        e = jnp.exp(sh - mx)
        ah = e / jnp.sum(e, axis=-1, keepdims=True)
        parts.append(jnp.dot(ah, kh, preferred_element_type=jnp.float32))
    ro = jnp.concatenate(parts, axis=-1)

    out_ref[0] = jnp.dot(ro, wout_ref[...], preferred_element_type=jnp.float32)

    nct = SEQ_TILE // CHUNK
    r = lax.broadcasted_iota(jnp.int32, (nct, SEQ_TILE), 0)
    c = lax.broadcasted_iota(jnp.int32, (nct, SEQ_TILE), 1)
    pool = jnp.where((c >> 4) == r, 1.0 / CHUNK, 0.0)
    cs_ref[0] = jnp.dot(pool, ro, preferred_element_type=jnp.float32)


def _prep_body(cs_ref, keys_ref, ww_ref, wg_ref, bg_ref, slots_ref,
               tidx_ref, wv_ref, scal_ref, d0_ref):
    ns = NUM_SLOTS
    cs = cs_ref[...]
    nrm = jnp.sqrt(jnp.sum(cs * cs, axis=-1, keepdims=True))
    csn = cs / jnp.maximum(nrm, 1e-12)
    keys = keys_ref[...]
    knrm = jnp.sqrt(jnp.sum(keys * keys, axis=-1, keepdims=True))
    kn = keys / jnp.maximum(knrm, 1e-12)
    ms = lax.dot_general(csn, kn, (((1,), (1,)), ((), ())),
                         preferred_element_type=jnp.float32)
    iota = lax.broadcasted_iota(jnp.int32, ms.shape, 1)
    for k in range(TOP_K):
        mx = jnp.max(ms, axis=-1, keepdims=True)
        eq = ms == mx
        ik = jnp.min(jnp.where(eq, iota, ns), axis=-1, keepdims=True)
        tidx_ref[:, k:k + 1] = ik
        ms = jnp.where(iota == ik, -1e30, ms)

    wv = jnp.dot(cs, ww_ref[...], preferred_element_type=jnp.float32)
    wv_ref[...] = wv
    scal_ref[:, 0:1] = jnp.dot(wv, wg_ref[:SLOT_DIM, :],
                               preferred_element_type=jnp.float32)
    scal_ref[:, 1:2] = jnp.dot(wv, wg_ref[SLOT_DIM:, :],
                               preferred_element_type=jnp.float32) + bg_ref[0, 0]
    d0_ref[...] = jnp.dot(slots_ref[...], wg_ref[:SLOT_DIM, :],
                          preferred_element_type=jnp.float32)


def _gate_body(nchunks, d0_hbm, fidx_hbm, wv1_hbm, wv2_hbm, gates_hbm,
               d_v, fidx_v, wv1_v, wv2_v, g_v):
    cid = lax.axis_index("c")
    sid = lax.axis_index("s")

    @pl.when(jnp.logical_and(cid == 0, sid == 0))
    def _():
        pltpu.sync_copy(d0_hbm, d_v)
        pltpu.sync_copy(fidx_hbm, fidx_v)
        pltpu.sync_copy(wv1_hbm, wv1_v)
        pltpu.sync_copy(wv2_hbm, wv2_v)
        lane = lax.iota(jnp.int32, 16)
        mask4 = lane < 4

        def step(c, carry):
            i0 = fidx_v[c, 0]
            i1 = fidx_v[c, 1]
            i2 = fidx_v[c, 2]
            v0 = plsc.load_gather(d_v, [i0])
            v1 = plsc.load_gather(d_v, [i1])
            v2 = plsc.load_gather(d_v, [i2])
            s = (v0 + v1 + v2) * (1.0 / 3.0) + wv2_v[c]
            g = 1.0 / (1.0 + jnp.exp(-s))
            omg = 1.0 - g
            wv1c = g * wv1_v[c]
            plsc.store_scatter(d_v, [i0], omg * v0 + wv1c, mask=mask4)
            plsc.store_scatter(d_v, [i1], omg * v1 + wv1c, mask=mask4)
            plsc.store_scatter(d_v, [i2], omg * v2 + wv1c, mask=mask4)
            g_v[c] = g
            return carry

        lax.fori_loop(0, nchunks, step, 0)
        pltpu.sync_copy(g_v, gates_hbm)


def _final_body(gates_ref, tidx_ref, wv_ref, slots_ref, lng_ref, lnb_ref,
                out_ref):
    nb = out_ref.shape[0]
    nc = gates_ref.shape[0]
    ns = NUM_SLOTS
    u = jnp.where(
        lax.broadcasted_iota(jnp.int32, (nc, nc), 0)
        < lax.broadcasted_iota(jnp.int32, (nc, nc), 1), 1.0, 0.0)
    ins = lax.broadcasted_iota(jnp.int32, (nc, ns), 1)
    ones_c = jnp.ones((nc, 1), jnp.float32)
    slots0 = slots_ref[...]
    for b in range(nb):
        g_col = gates_ref[:, b:b + 1]
        msk = jnp.zeros((nc, ns), jnp.float32)
        for k in range(TOP_K):
            idx = tidx_ref[b * nc:(b + 1) * nc, k:k + 1]
            msk = msk + jnp.where(ins == idx, 1.0, 0.0)
        t = 1.0 - g_col * msk
        lt = jnp.log(jnp.maximum(t, 1e-30))
        rsum = jnp.dot(u, lt, preferred_element_type=jnp.float32)
        lsuf = jnp.exp(rsum)
        a_col = jnp.exp(lax.dot_general(lt, ones_c, (((0,), (0,)), ((), ())),
                                        preferred_element_type=jnp.float32))
        w = g_col * msk * lsuf
        wv_b = wv_ref[b * nc:(b + 1) * nc, :]
        contrib = lax.dot_general(w, wv_b, (((0,), (0,)), ((), ())),
                                  preferred_element_type=jnp.float32)
        sl = a_col * slots0 + contrib
        m = jnp.mean(sl, axis=-1, keepdims=True)
        xc = sl - m
        v = jnp.mean(xc * xc, axis=-1, keepdims=True)
        out_ref[b] = xc * lax.rsqrt(v + 1e-5) * lng_ref[...] + lnb_ref[...]


def kernel(x, slot_memory, slot_keys, W_in, ln_in_g, ln_in_b, W_write, Wg,
           bg, W_out, ln_s_g, ln_s_b):
    B, S, E = x.shape
    Ns, D = slot_keys.shape
    C = S // CHUNK
    nt = S // SEQ_TILE
    nct = SEQ_TILE // CHUNK
    slots0 = slot_memory[0]

    out, cs = pl.pallas_call(
        _dense_body,
        grid=(B, nt),
        in_specs=[
            pl.BlockSpec((1, SEQ_TILE, E), lambda b, t: (b, t, 0)),
            pl.BlockSpec((1, E), lambda b, t: (0, 0)),
            pl.BlockSpec((1, E), lambda b, t: (0, 0)),
            pl.BlockSpec((E, D), lambda b, t: (0, 0)),
            pl.BlockSpec((Ns, D), lambda b, t: (0, 0)),
            pl.BlockSpec((D, E), lambda b, t: (0, 0)),
        ],
        out_specs=[
            pl.BlockSpec((1, SEQ_TILE, E), lambda b, t: (b, t, 0)),
            pl.BlockSpec((1, nct, D), lambda b, t: (b, t, 0)),
        ],
        out_shape=[
            jax.ShapeDtypeStruct((B, S, E), jnp.float32),
            jax.ShapeDtypeStruct((B, C, D), jnp.float32),
        ],
        compiler_params=pltpu.CompilerParams(
            dimension_semantics=("parallel", "arbitrary")),
    )(x, ln_in_g.reshape(1, E), ln_in_b.reshape(1, E), W_in, slots0, W_out)

    tidx, wv, scal, d0 = pl.pallas_call(
        _prep_body,
        out_shape=[
            jax.ShapeDtypeStruct((B * C, TOP_K), jnp.int32),
            jax.ShapeDtypeStruct((B * C, D), jnp.float32),
            jax.ShapeDtypeStruct((B * C, 2), jnp.float32),
            jax.ShapeDtypeStruct((Ns, 1), jnp.float32),
        ],
    )(cs.reshape(B * C, D), slot_keys, W_write, Wg, bg.reshape(1, 1), slots0)

    t3 = jnp.transpose(tidx.reshape(B, C, TOP_K), (1, 2, 0))
    fidx = t3 + (jnp.arange(B, dtype=jnp.int32) * Ns)[None, None, :]
    fidx = jnp.pad(fidx, ((0, 0), (0, 0), (0, 16 - B)))
    sc2 = scal.reshape(B, C, 2)
    wv1v = jnp.pad(jnp.transpose(sc2[..., 0], (1, 0)), ((0, 0), (0, 16 - B)))
    wv2v = jnp.pad(jnp.transpose(sc2[..., 1], (1, 0)), ((0, 0), (0, 16 - B)))
    d0flat = jnp.tile(d0[:, 0], B)

    gates = pl.kernel(
        functools.partial(_gate_body, C),
        out_type=jax.ShapeDtypeStruct((C, 16), jnp.float32),
        mesh=plsc.VectorSubcoreMesh(core_axis_name="c", subcore_axis_name="s",
                                    num_cores=2, num_subcores=16),
        scratch_types=[
            pltpu.VMEM((B * Ns,), jnp.float32),
            pltpu.VMEM((C, TOP_K, 16), jnp.int32),
            pltpu.VMEM((C, 16), jnp.float32),
            pltpu.VMEM((C, 16), jnp.float32),
            pltpu.VMEM((C, 16), jnp.float32),
        ],
        compiler_params=pltpu.CompilerParams(needs_layout_passes=False,
                                             use_tc_tiling_on_sc=False),
    )(d0flat, fidx, wv1v, wv2v)

    new_slots = pl.pallas_call(
        _final_body,
        out_shape=jax.ShapeDtypeStruct((B, Ns, D), jnp.float32),
    )(gates, tidx, wv, slots0, ln_s_g.reshape(1, D), ln_s_b.reshape(1, D))

    return out, new_slots
